# baseline (speedup 1.0000x reference)
import functools

import jax
import jax.numpy as jnp
from jax import lax
from jax.experimental import pallas as pl
from jax.experimental.pallas import tpu as pltpu
from jax.experimental.pallas import tpu_sc as plsc

_N = 10000
_E = 320000
_D = 128
_H1, _C1 = 8, 16
_C2 = 32

_NP = 10112
_NTILES = 16
_RPT = _NP // _NTILES
_NW = 32
_EED = _E + _N
_K1, _CH1 = 64, 168
_K2, _CH2 = 128, 88
_EP1 = _K1 * _CH1 * _NW
_EP2 = _K2 * _CH2 * _NW
_TROW = 16


def _lrelu(t):
    return jnp.where(t >= 0, t, t * jnp.float32(0.2))


_GDN = lax.GatherDimensionNumbers(
    offset_dims=(), collapsed_slice_dims=(0,), start_index_map=(0,))


def _vgather(v, idx):
    return lax.gather(v, idx[:, None], _GDN, slice_sizes=(1,),
                      mode=lax.GatherScatterMode.PROMISE_IN_BOUNDS)


def _tc_pre1_body(x_ref, w1_ref, asm_ref, adm_ref, h_ref, asp_ref, adb_ref):
    h = jnp.dot(x_ref[...], w1_ref[...], preferred_element_type=jnp.float32)
    h_ref[...] = h
    as1 = jnp.dot(h, asm_ref[...], preferred_element_type=jnp.float32)
    ad1 = jnp.dot(h, adm_ref[...], preferred_element_type=jnp.float32)
    z = jnp.zeros((_N, _H1), dtype=jnp.float32)
    asp_ref[...] = jnp.concatenate([as1, z], axis=1)
    bb = _lrelu(jnp.max(as1, axis=0, keepdims=True) + ad1)
    adb_ref[...] = jnp.concatenate([ad1, bb], axis=1)


def _tc_mid_body(accw_ref, accp_ref, b1_ref, rep_ref, w2_ref, as2_ref,
                 ad2_ref, h2_ref, asp2_ref, adb2_ref):
    num = (accw_ref[0] + accw_ref[1])[:_N]
    ph = (accp_ref[0] + accp_ref[1])[:_N]
    den = jnp.dot(ph, rep_ref[...], preferred_element_type=jnp.float32)
    o1 = num / (den + jnp.float32(1e-16)) + b1_ref[...]
    act = jnp.where(o1 > 0, o1, jnp.exp(o1) - 1)
    h2 = jnp.dot(act, w2_ref[...], preferred_element_type=jnp.float32)
    h2_ref[...] = h2
    as2 = jnp.dot(h2, as2_ref[...], preferred_element_type=jnp.float32)
    ad2 = jnp.dot(h2, ad2_ref[...], preferred_element_type=jnp.float32)
    bb2 = _lrelu(jnp.max(as2, axis=0, keepdims=True) + ad2)
    asp2_ref[...] = jnp.concatenate(
        [as2, jnp.zeros((_N, _TROW - 1), dtype=jnp.float32)], axis=1)
    adb2_ref[...] = jnp.concatenate(
        [ad2, bb2, jnp.zeros((_N, _TROW - 2), dtype=jnp.float32)], axis=1)


def _tc_post2_body(accw_ref, accp_ref, b2_ref, out_ref):
    num = (accw_ref[0] + accw_ref[1])[:_N]
    den = (accp_ref[0] + accp_ref[1])[:_N, :1]
    o = num / (den + jnp.float32(1e-16)) + b2_ref[...]
    m = jnp.max(o, axis=1, keepdims=True)
    e = o - m
    out_ref[...] = e - jnp.log(jnp.sum(jnp.exp(e), axis=1, keepdims=True))


def _make_sc_edge(hw, heads, ch, K, CH):
    nvec = hw // 16
    row = 8 + hw
    chb = CH // 4
    mesh = plsc.VectorSubcoreMesh(core_axis_name="c", subcore_axis_name="s")

    @functools.partial(
        pl.kernel, mesh=mesh,
        compiler_params=pltpu.CompilerParams(use_tc_tiling_on_sc=False),
        out_type=[
            jax.ShapeDtypeStruct((2, _NP, hw), jnp.float32),
            jax.ShapeDtypeStruct((2, _NP, 8), jnp.float32),
        ],
        scratch_types=[
            pltpu.VMEM((2, 4, 3, K), jnp.int32),
            pltpu.VMEM((2, K, hw), jnp.float32),
            pltpu.VMEM((2, K, _TROW), jnp.float32),
            pltpu.VMEM((2, K, _TROW), jnp.float32),
            pltpu.VMEM((2, K, row), jnp.float32),
            pltpu.VMEM_SHARED((_NP, row), jnp.float32),
            pltpu.SemaphoreType.DMA((2,)),
            pltpu.SemaphoreType.DMA((2,)),
            pltpu.SemaphoreType.DMA((2,)),
        ],
    )
    def sc_edge(h_hbm, asp_hbm, adb_hbm, eidx_hbm, ow_hbm, op_hbm,
                sidx, hrows, arows, drows, orow, accum, si, sg, so):
        c = lax.axis_index("c")
        s = lax.axis_index("s")
        wid = s * 2 + c
        r0 = pl.multiple_of(s * _RPT, 8)
        lanes = jnp.arange(16, dtype=jnp.int32)
        bidx = (lanes & (heads - 1)) + heads
        m8 = lanes < 8
        bbase = wid * chb
        zv = jnp.zeros((16,), dtype=jnp.float32)

        @plsc.parallel_loop(0, K, 1)
        def _zero(k):
            orow[0, k, pl.ds(0, 16)] = zv
            for j in range(nvec):
                orow[0, k, pl.ds(8 + 16 * j, 16)] = zv

        for i in range(_RPT // K):
            pltpu.sync_copy(orow.at[0], accum.at[pl.ds(r0 + i * K, K)])
        _rem = _RPT - (_RPT // K) * K
        if _rem:
            pltpu.sync_copy(orow.at[0, pl.ds(0, _rem)],
                            accum.at[pl.ds(r0 + _RPT - _rem, _rem)])
        plsc.subcore_barrier()

        def issue_idx(blk, u):
            pltpu.async_copy(eidx_hbm.at[bbase + blk], sidx.at[u], si.at[u])

        def wait_idx(blk, u):
            pltpu.make_async_copy(eidx_hbm.at[bbase + blk], sidx.at[u],
                                  si.at[u]).wait()

        def issue_gather(u, j, b):
            pltpu.async_copy(h_hbm.at[sidx.at[u, j, 0]], hrows.at[b],
                             sg.at[b])
            pltpu.async_copy(asp_hbm.at[sidx.at[u, j, 0]], arows.at[b],
                             sg.at[b])
            pltpu.async_copy(adb_hbm.at[sidx.at[u, j, 1]], drows.at[b],
                             sg.at[b])

        def wait_gather(u, j, b):
            pltpu.make_async_copy(h_hbm.at[sidx.at[u, j, 0]], hrows.at[b],
                                  sg.at[b]).wait()
            pltpu.make_async_copy(asp_hbm.at[sidx.at[u, j, 0]],
                                  arows.at[b], sg.at[b]).wait()
            pltpu.make_async_copy(adb_hbm.at[sidx.at[u, j, 1]],
                                  drows.at[b], sg.at[b]).wait()

        def issue_scatter(u, j, b):
            pltpu.async_copy(orow.at[b], accum.at[sidx.at[u, j, 2]],
                             so.at[b], add=True)

        def wait_scatter(u, j, b):
            pltpu.make_async_copy(orow.at[b], accum.at[sidx.at[u, j, 2]],
                                  so.at[b]).wait()

        def compute(b):
            @plsc.parallel_loop(0, K, 1, unroll=8)
            def edge_body(k):
                va = arows[b, k, pl.ds(0, 16)]
                vd = drows[b, k, pl.ds(0, 16)]
                u = _lrelu(va + vd)
                bsh = _vgather(vd, bidx)
                pv = jnp.exp(u - bsh)
                w0 = hrows[b, k, pl.ds(0, 16)] * _vgather(
                    pv, jnp.zeros((16,), dtype=jnp.int32))
                front = jnp.where(m8, pv, _vgather(w0, lanes & 7))
                orow[b, k, pl.ds(0, 16)] = front
                orow[b, k, pl.ds(8, 16)] = w0
                for j in range(1, nvec):
                    hj = hrows[b, k, pl.ds(16 * j, 16)]
                    pj = _vgather(pv, jnp.full((16,), (16 * j) // ch,
                                               dtype=jnp.int32))
                    orow[b, k, pl.ds(8 + 16 * j, 16)] = hj * pj

        pltpu.sync_copy(eidx_hbm.at[bbase], sidx.at[0])
        issue_gather(0, 0, 0)
        issue_idx(1, 1)

        def window_body(g, carry):
            for pos in range(8):
                ci = g * 8 + pos
                b = pos % 2
                u_cur = (pos // 4) % 2
                u_nxt = ((pos + 1) // 4) % 2
                u_pv2 = ((pos - 2) // 4) % 2

                @pl.when(ci + 1 < CH)
                def _():
                    if pos % 4 == 3:
                        wait_idx((ci + 1) // 4, u_nxt)
                    issue_gather(u_nxt, (pos + 1) % 4, 1 - b)

                wait_gather(u_cur, pos % 4, b)

                @pl.when(ci >= 2)
                def _():
                    wait_scatter(u_pv2, (pos - 2) % 4, b)

                if pos % 4 == 1:
                    blk_t = (ci + 3) // 4

                    @pl.when((ci >= 5) & (blk_t < chb))
                    def _():
                        issue_idx(blk_t, 1 if pos == 1 else 0)

                compute(b)
                issue_scatter(u_cur, pos % 4, b)
            return carry

        lax.fori_loop(0, CH // 8, window_body, 0, unroll=False)
        _ub = ((CH - 2) // 4) % 2
        wait_scatter(_ub, 2, 0)
        wait_scatter(_ub, 3, 1)
        plsc.subcore_barrier()
        pltpu.sync_copy(accum.at[pl.ds(r0, _RPT), pl.ds(8, hw)],
                        ow_hbm.at[c, pl.ds(r0, _RPT)])
        pltpu.sync_copy(accum.at[pl.ds(r0, _RPT), pl.ds(0, 8)],
                        op_hbm.at[c, pl.ds(r0, _RPT)])

    return sc_edge


_sc_edge1 = _make_sc_edge(_D, _H1, _C1, _K1, _CH1)
_sc_edge2 = _make_sc_edge(_C2, 1, _C2, _K2, _CH2)


def _block_diag(a):
    heads, ch = a.shape
    eye = jnp.eye(heads, dtype=jnp.float32)
    return (a[:, :, None] * eye[:, None, :]).reshape(heads * ch, heads)


def _build_eidx(e0, e1, K, EP):
    loops = jnp.arange(_N, dtype=jnp.int32)
    npad = EP - _EED
    ar = jnp.arange(npad, dtype=jnp.int32)
    pad_g = ar % _N
    pad_s = _N + (ar % 16)
    gsrc = jnp.concatenate([e0, loops, pad_g])
    gdst = jnp.concatenate([e1, loops, pad_g])
    sdst = jnp.concatenate([e1, loops, pad_s])
    ncht = EP // K
    st = jnp.stack([gsrc.reshape(ncht, K), gdst.reshape(ncht, K),
                    sdst.reshape(ncht, K)], axis=1)
    return st.reshape(ncht // 4, 4, 3, K)


def kernel(x, edge_index, W1, a_src1, a_dst1, b1, W2, a_src2, a_dst2, b2):
    eidx1 = _build_eidx(edge_index[0], edge_index[1], _K1, _EP1)
    eidx2 = _build_eidx(edge_index[0], edge_index[1], _K2, _EP2)

    asm1 = _block_diag(a_src1)
    adm1 = _block_diag(a_dst1)
    h1, asp1, adb1 = pl.pallas_call(
        _tc_pre1_body,
        out_shape=[
            jax.ShapeDtypeStruct((_N, _D), jnp.float32),
            jax.ShapeDtypeStruct((_N, _TROW), jnp.float32),
            jax.ShapeDtypeStruct((_N, _TROW), jnp.float32),
        ],
    )(x, W1, asm1, adm1)

    accw1, accp1 = _sc_edge1(h1, asp1, adb1, eidx1)

    rep = jnp.repeat(jnp.eye(_H1, dtype=jnp.float32), _C1, axis=1)
    h2, asp2, adb2 = pl.pallas_call(
        _tc_mid_body,
        out_shape=[
            jax.ShapeDtypeStruct((_N, _C2), jnp.float32),
            jax.ShapeDtypeStruct((_N, _TROW), jnp.float32),
            jax.ShapeDtypeStruct((_N, _TROW), jnp.float32),
        ],
    )(accw1, accp1, b1.reshape(1, _D), rep, W2, a_src2.reshape(_C2, 1),
      a_dst2.reshape(_C2, 1))

    accw2, accp2 = _sc_edge2(h2, asp2, adb2, eidx2)

    out = pl.pallas_call(
        _tc_post2_body,
        out_shape=jax.ShapeDtypeStruct((_N, _C2), jnp.float32),
    )(accw2, accp2, b2.reshape(1, _C2))
    return out


# speedup vs baseline: 1.0886x; 1.0886x over previous
import functools

import jax
import jax.numpy as jnp
from jax import lax
from jax.experimental import pallas as pl
from jax.experimental.pallas import tpu as pltpu
from jax.experimental.pallas import tpu_sc as plsc

_N = 10000
_E = 320000
_D = 128
_H1, _C1 = 8, 16
_C2 = 32

_NP = 10112
_NTILES = 16
_RPT = _NP // _NTILES
_NW = 32
_EED = _E + _N
_K1, _CH1 = 64, 168
_K2, _CH2 = 128, 88
_EP1 = _K1 * _CH1 * _NW
_EP2 = _K2 * _CH2 * _NW
_TROW = 16


def _lrelu(t):
    return jnp.where(t >= 0, t, t * jnp.float32(0.2))


_GDN = lax.GatherDimensionNumbers(
    offset_dims=(), collapsed_slice_dims=(0,), start_index_map=(0,))


def _vgather(v, idx):
    return lax.gather(v, idx[:, None], _GDN, slice_sizes=(1,),
                      mode=lax.GatherScatterMode.PROMISE_IN_BOUNDS)


def _tc_pre1_body(x_ref, w1_ref, asm_ref, adm_ref, h_ref, asp_ref, adb_ref):
    h = jnp.dot(x_ref[...], w1_ref[...], preferred_element_type=jnp.float32)
    h_ref[...] = h
    as1 = jnp.dot(h, asm_ref[...], preferred_element_type=jnp.float32)
    ad1 = jnp.dot(h, adm_ref[...], preferred_element_type=jnp.float32)
    z = jnp.zeros((_N, _H1), dtype=jnp.float32)
    asp_ref[...] = jnp.concatenate([as1, z], axis=1)
    bb = _lrelu(jnp.max(as1, axis=0, keepdims=True) + ad1)
    adb_ref[...] = jnp.concatenate([ad1, bb], axis=1)


def _tc_mid_body(accw_ref, accp_ref, b1_ref, rep_ref, w2_ref, as2_ref,
                 ad2_ref, h2_ref, asp2_ref, adb2_ref):
    num = (accw_ref[0] + accw_ref[1])[:_N]
    ph = (accp_ref[0] + accp_ref[1])[:_N]
    den = jnp.dot(ph, rep_ref[...], preferred_element_type=jnp.float32)
    o1 = num / (den + jnp.float32(1e-16)) + b1_ref[...]
    act = jnp.where(o1 > 0, o1, jnp.exp(o1) - 1)
    h2 = jnp.dot(act, w2_ref[...], preferred_element_type=jnp.float32)
    h2_ref[...] = h2
    as2 = jnp.dot(h2, as2_ref[...], preferred_element_type=jnp.float32)
    ad2 = jnp.dot(h2, ad2_ref[...], preferred_element_type=jnp.float32)
    bb2 = _lrelu(jnp.max(as2, axis=0, keepdims=True) + ad2)
    asp2_ref[...] = jnp.concatenate(
        [as2, jnp.zeros((_N, _TROW - 1), dtype=jnp.float32)], axis=1)
    adb2_ref[...] = jnp.concatenate(
        [ad2, bb2, jnp.zeros((_N, _TROW - 2), dtype=jnp.float32)], axis=1)


def _tc_post2_body(accw_ref, accp_ref, b2_ref, out_ref):
    num = (accw_ref[0] + accw_ref[1])[:_N]
    den = (accp_ref[0] + accp_ref[1])[:_N, :1]
    o = num / (den + jnp.float32(1e-16)) + b2_ref[...]
    m = jnp.max(o, axis=1, keepdims=True)
    e = o - m
    out_ref[...] = e - jnp.log(jnp.sum(jnp.exp(e), axis=1, keepdims=True))


def _make_sc_edge(hw, heads, ch, K, CH):
    nvec = hw // 16
    row = 8 + hw
    chb = CH // 4
    mesh = plsc.VectorSubcoreMesh(core_axis_name="c", subcore_axis_name="s")

    @functools.partial(
        pl.kernel, mesh=mesh,
        compiler_params=pltpu.CompilerParams(use_tc_tiling_on_sc=False),
        out_type=[
            jax.ShapeDtypeStruct((2, _NP, hw), jnp.float32),
            jax.ShapeDtypeStruct((2, _NP, 8), jnp.float32),
        ],
        scratch_types=[
            pltpu.VMEM((2, 4, 3, K), jnp.int32),
            pltpu.VMEM((2, K, hw), jnp.float32),
            pltpu.VMEM((2, K, _TROW), jnp.float32),
            pltpu.VMEM((2, K, _TROW), jnp.float32),
            pltpu.VMEM((2, K, row), jnp.float32),
            pltpu.VMEM_SHARED((_NP, row), jnp.float32),
            pltpu.SemaphoreType.DMA((2,)),
            pltpu.SemaphoreType.DMA((2,)),
            pltpu.SemaphoreType.DMA((2,)),
        ],
    )
    def sc_edge(h_hbm, asp_hbm, adb_hbm, eidx_hbm, ow_hbm, op_hbm,
                sidx, hrows, arows, drows, orow, accum, si, sg, so):
        c = lax.axis_index("c")
        s = lax.axis_index("s")
        wid = s * 2 + c
        r0 = pl.multiple_of(s * _RPT, 8)
        lanes = jnp.arange(16, dtype=jnp.int32)
        bidx = (lanes & (heads - 1)) + heads
        m8 = lanes < 8
        bbase = wid * chb
        zv = jnp.zeros((16,), dtype=jnp.float32)

        @plsc.parallel_loop(0, K, 1)
        def _zero(k):
            orow[0, k, pl.ds(0, 16)] = zv
            for j in range(nvec):
                orow[0, k, pl.ds(8 + 16 * j, 16)] = zv

        for i in range(_RPT // K):
            pltpu.sync_copy(orow.at[0], accum.at[pl.ds(r0 + i * K, K)])
        _rem = _RPT - (_RPT // K) * K
        if _rem:
            pltpu.sync_copy(orow.at[0, pl.ds(0, _rem)],
                            accum.at[pl.ds(r0 + _RPT - _rem, _rem)])
        plsc.subcore_barrier()

        def issue_idx(blk, u):
            pltpu.async_copy(eidx_hbm.at[bbase + blk], sidx.at[u], si.at[u])

        def wait_idx(blk, u):
            pltpu.make_async_copy(eidx_hbm.at[bbase + blk], sidx.at[u],
                                  si.at[u]).wait()

        def issue_gather(u, j, b):
            pltpu.async_copy(h_hbm.at[sidx.at[u, j, 0]], hrows.at[b],
                             sg.at[b])
            pltpu.async_copy(asp_hbm.at[sidx.at[u, j, 0]], arows.at[b],
                             sg.at[b])
            pltpu.async_copy(adb_hbm.at[sidx.at[u, j, 1]], drows.at[b],
                             sg.at[b])

        def wait_gather(u, j, b):
            pltpu.make_async_copy(h_hbm.at[sidx.at[u, j, 0]], hrows.at[b],
                                  sg.at[b]).wait()
            pltpu.make_async_copy(asp_hbm.at[sidx.at[u, j, 0]],
                                  arows.at[b], sg.at[b]).wait()
            pltpu.make_async_copy(adb_hbm.at[sidx.at[u, j, 1]],
                                  drows.at[b], sg.at[b]).wait()

        def issue_scatter(u, j, b):
            pltpu.async_copy(orow.at[b], accum.at[sidx.at[u, j, 2]],
                             so.at[b], add=True)

        def wait_scatter(u, j, b):
            pltpu.make_async_copy(orow.at[b], accum.at[sidx.at[u, j, 2]],
                                  so.at[b]).wait()

        def compute(b):
            @plsc.parallel_loop(0, K, 1, unroll=2)
            def edge_body(k):
                va = arows[b, k, pl.ds(0, 16)]
                vd = drows[b, k, pl.ds(0, 16)]
                u = _lrelu(va + vd)
                bsh = _vgather(vd, bidx)
                pv = jnp.exp(u - bsh)
                w0 = hrows[b, k, pl.ds(0, 16)] * _vgather(
                    pv, jnp.zeros((16,), dtype=jnp.int32))
                front = jnp.where(m8, pv, _vgather(w0, lanes & 7))
                orow[b, k, pl.ds(0, 16)] = front
                orow[b, k, pl.ds(8, 16)] = w0
                for j in range(1, nvec):
                    hj = hrows[b, k, pl.ds(16 * j, 16)]
                    pj = _vgather(pv, jnp.full((16,), (16 * j) // ch,
                                               dtype=jnp.int32))
                    orow[b, k, pl.ds(8 + 16 * j, 16)] = hj * pj

        pltpu.sync_copy(eidx_hbm.at[bbase], sidx.at[0])
        issue_gather(0, 0, 0)
        issue_idx(1, 1)

        def window_body(g, carry):
            for pos in range(8):
                ci = g * 8 + pos
                b = pos % 2
                u_cur = (pos // 4) % 2
                u_nxt = ((pos + 1) // 4) % 2
                u_pv2 = ((pos - 2) // 4) % 2

                @pl.when(ci + 1 < CH)
                def _():
                    if pos % 4 == 3:
                        wait_idx((ci + 1) // 4, u_nxt)
                    issue_gather(u_nxt, (pos + 1) % 4, 1 - b)

                wait_gather(u_cur, pos % 4, b)

                @pl.when(ci >= 2)
                def _():
                    wait_scatter(u_pv2, (pos - 2) % 4, b)

                if pos % 4 == 1:
                    blk_t = (ci + 3) // 4

                    @pl.when((ci >= 5) & (blk_t < chb))
                    def _():
                        issue_idx(blk_t, 1 if pos == 1 else 0)

                compute(b)
                issue_scatter(u_cur, pos % 4, b)
            return carry

        lax.fori_loop(0, CH // 8, window_body, 0, unroll=False)
        _ub = ((CH - 2) // 4) % 2
        wait_scatter(_ub, 2, 0)
        wait_scatter(_ub, 3, 1)
        plsc.subcore_barrier()
        pltpu.sync_copy(accum.at[pl.ds(r0, _RPT), pl.ds(8, hw)],
                        ow_hbm.at[c, pl.ds(r0, _RPT)])
        pltpu.sync_copy(accum.at[pl.ds(r0, _RPT), pl.ds(0, 8)],
                        op_hbm.at[c, pl.ds(r0, _RPT)])

    return sc_edge


_sc_edge1 = _make_sc_edge(_D, _H1, _C1, _K1, _CH1)
_sc_edge2 = _make_sc_edge(_C2, 1, _C2, _K2, _CH2)


def _block_diag(a):
    heads, ch = a.shape
    eye = jnp.eye(heads, dtype=jnp.float32)
    return (a[:, :, None] * eye[:, None, :]).reshape(heads * ch, heads)


def _build_eidx(e0, e1, K, EP):
    loops = jnp.arange(_N, dtype=jnp.int32)
    npad = EP - _EED
    ar = jnp.arange(npad, dtype=jnp.int32)
    pad_g = ar % _N
    pad_s = _N + (ar % 16)
    gsrc = jnp.concatenate([e0, loops, pad_g])
    gdst = jnp.concatenate([e1, loops, pad_g])
    sdst = jnp.concatenate([e1, loops, pad_s])
    ncht = EP // K
    st = jnp.stack([gsrc.reshape(ncht, K), gdst.reshape(ncht, K),
                    sdst.reshape(ncht, K)], axis=1)
    return st.reshape(ncht // 4, 4, 3, K)


def kernel(x, edge_index, W1, a_src1, a_dst1, b1, W2, a_src2, a_dst2, b2):
    eidx1 = _build_eidx(edge_index[0], edge_index[1], _K1, _EP1)
    eidx2 = _build_eidx(edge_index[0], edge_index[1], _K2, _EP2)

    asm1 = _block_diag(a_src1)
    adm1 = _block_diag(a_dst1)
    h1, asp1, adb1 = pl.pallas_call(
        _tc_pre1_body,
        out_shape=[
            jax.ShapeDtypeStruct((_N, _D), jnp.float32),
            jax.ShapeDtypeStruct((_N, _TROW), jnp.float32),
            jax.ShapeDtypeStruct((_N, _TROW), jnp.float32),
        ],
    )(x, W1, asm1, adm1)

    accw1, accp1 = _sc_edge1(h1, asp1, adb1, eidx1)

    rep = jnp.repeat(jnp.eye(_H1, dtype=jnp.float32), _C1, axis=1)
    h2, asp2, adb2 = pl.pallas_call(
        _tc_mid_body,
        out_shape=[
            jax.ShapeDtypeStruct((_N, _C2), jnp.float32),
            jax.ShapeDtypeStruct((_N, _TROW), jnp.float32),
            jax.ShapeDtypeStruct((_N, _TROW), jnp.float32),
        ],
    )(accw1, accp1, b1.reshape(1, _D), rep, W2, a_src2.reshape(_C2, 1),
      a_dst2.reshape(_C2, 1))

    accw2, accp2 = _sc_edge2(h2, asp2, adb2, eidx2)

    out = pl.pallas_call(
        _tc_post2_body,
        out_shape=jax.ShapeDtypeStruct((_N, _C2), jnp.float32),
    )(accw2, accp2, b2.reshape(1, _C2))
    return out


# speedup vs baseline: 1.1020x; 1.0123x over previous
import functools

import jax
import jax.numpy as jnp
from jax import lax
from jax.experimental import pallas as pl
from jax.experimental.pallas import tpu as pltpu
from jax.experimental.pallas import tpu_sc as plsc

_N = 10000
_E = 320000
_D = 128
_H1, _C1 = 8, 16
_C2 = 32

_NP = 10112
_NTILES = 16
_RPT = _NP // _NTILES
_NW = 32
_EED = _E + _N
_K1, _CH1 = 64, 168
_K2, _CH2 = 128, 88
_EP1 = _K1 * _CH1 * _NW
_EP2 = _K2 * _CH2 * _NW
_TROW = 16


def _lrelu(t):
    return jnp.where(t >= 0, t, t * jnp.float32(0.2))


_GDN = lax.GatherDimensionNumbers(
    offset_dims=(), collapsed_slice_dims=(0,), start_index_map=(0,))


def _vgather(v, idx):
    return lax.gather(v, idx[:, None], _GDN, slice_sizes=(1,),
                      mode=lax.GatherScatterMode.PROMISE_IN_BOUNDS)


def _tc_pre1_body(x_ref, w1_ref, asm_ref, adm_ref, h_ref, asp_ref, adb_ref):
    h = jnp.dot(x_ref[...], w1_ref[...], preferred_element_type=jnp.float32)
    h_ref[...] = h
    as1 = jnp.dot(h, asm_ref[...], preferred_element_type=jnp.float32)
    ad1 = jnp.dot(h, adm_ref[...], preferred_element_type=jnp.float32)
    z = jnp.zeros((_N, _H1), dtype=jnp.float32)
    asp_ref[...] = jnp.concatenate([as1, z], axis=1)
    bb = _lrelu(jnp.max(as1, axis=0, keepdims=True) + ad1)
    adb_ref[...] = jnp.concatenate([ad1, bb], axis=1)


def _tc_mid_body(accw_ref, accp_ref, b1_ref, rep_ref, w2_ref, as2_ref,
                 ad2_ref, h2_ref, asp2_ref, adb2_ref):
    num = (accw_ref[0] + accw_ref[1])[:_N]
    ph = (accp_ref[0] + accp_ref[1])[:_N]
    den = jnp.dot(ph, rep_ref[...], preferred_element_type=jnp.float32)
    o1 = num / (den + jnp.float32(1e-16)) + b1_ref[...]
    act = jnp.where(o1 > 0, o1, jnp.exp(o1) - 1)
    h2 = jnp.dot(act, w2_ref[...], preferred_element_type=jnp.float32)
    h2_ref[...] = h2
    as2 = jnp.dot(h2, as2_ref[...], preferred_element_type=jnp.float32)
    ad2 = jnp.dot(h2, ad2_ref[...], preferred_element_type=jnp.float32)
    bb2 = _lrelu(jnp.max(as2, axis=0, keepdims=True) + ad2)
    asp2_ref[...] = jnp.concatenate(
        [as2, jnp.zeros((_N, _TROW - 1), dtype=jnp.float32)], axis=1)
    adb2_ref[...] = jnp.concatenate(
        [ad2, bb2, jnp.zeros((_N, _TROW - 2), dtype=jnp.float32)], axis=1)


def _tc_post2_body(accw_ref, accp_ref, b2_ref, out_ref):
    num = (accw_ref[0] + accw_ref[1])[:_N]
    den = (accp_ref[0] + accp_ref[1])[:_N, :1]
    o = num / (den + jnp.float32(1e-16)) + b2_ref[...]
    m = jnp.max(o, axis=1, keepdims=True)
    e = o - m
    out_ref[...] = e - jnp.log(jnp.sum(jnp.exp(e), axis=1, keepdims=True))


def _make_sc_edge(hw, heads, ch, K, CH):
    nvec = hw // 16
    row = 8 + hw
    chb = CH // 4
    mesh = plsc.VectorSubcoreMesh(core_axis_name="c", subcore_axis_name="s")

    @functools.partial(
        pl.kernel, mesh=mesh,
        compiler_params=pltpu.CompilerParams(use_tc_tiling_on_sc=False),
        out_type=[
            jax.ShapeDtypeStruct((2, _NP, hw), jnp.float32),
            jax.ShapeDtypeStruct((2, _NP, 8), jnp.float32),
        ],
        scratch_types=[
            pltpu.VMEM((2, 4, 3, K), jnp.int32),
            pltpu.VMEM((2, K, hw), jnp.float32),
            pltpu.VMEM((2, K, _TROW), jnp.float32),
            pltpu.VMEM((2, K, _TROW), jnp.float32),
            pltpu.VMEM((2, K, row), jnp.float32),
            pltpu.VMEM_SHARED((_NP, row), jnp.float32),
            pltpu.SemaphoreType.DMA((2,)),
            pltpu.SemaphoreType.DMA((2,)),
            pltpu.SemaphoreType.DMA((2,)),
        ],
    )
    def sc_edge(h_hbm, asp_hbm, adb_hbm, eidx_hbm, ow_hbm, op_hbm,
                sidx, hrows, arows, drows, orow, accum, si, sg, so):
        c = lax.axis_index("c")
        s = lax.axis_index("s")
        wid = s * 2 + c
        r0 = pl.multiple_of(s * _RPT, 8)
        lanes = jnp.arange(16, dtype=jnp.int32)
        bidx = (lanes & (heads - 1)) + heads
        m8 = lanes < 8
        bbase = wid * chb
        zv = jnp.zeros((16,), dtype=jnp.float32)

        @plsc.parallel_loop(0, K, 1)
        def _zero(k):
            orow[0, k, pl.ds(0, 16)] = zv
            for j in range(nvec):
                orow[0, k, pl.ds(8 + 16 * j, 16)] = zv

        for i in range(_RPT // K):
            pltpu.sync_copy(orow.at[0], accum.at[pl.ds(r0 + i * K, K)])
        _rem = _RPT - (_RPT // K) * K
        if _rem:
            pltpu.sync_copy(orow.at[0, pl.ds(0, _rem)],
                            accum.at[pl.ds(r0 + _RPT - _rem, _rem)])
        plsc.subcore_barrier()

        def issue_idx(blk, u):
            pltpu.async_copy(eidx_hbm.at[bbase + blk], sidx.at[u], si.at[u])

        def wait_idx(blk, u):
            pltpu.make_async_copy(eidx_hbm.at[bbase + blk], sidx.at[u],
                                  si.at[u]).wait()

        def issue_gather(u, j, b):
            pltpu.async_copy(h_hbm.at[sidx.at[u, j, 0]], hrows.at[b],
                             sg.at[b])
            pltpu.async_copy(asp_hbm.at[sidx.at[u, j, 0]], arows.at[b],
                             sg.at[b])
            pltpu.async_copy(adb_hbm.at[sidx.at[u, j, 1]], drows.at[b],
                             sg.at[b])

        def wait_gather(u, j, b):
            pltpu.make_async_copy(h_hbm.at[sidx.at[u, j, 0]], hrows.at[b],
                                  sg.at[b]).wait()
            pltpu.make_async_copy(asp_hbm.at[sidx.at[u, j, 0]],
                                  arows.at[b], sg.at[b]).wait()
            pltpu.make_async_copy(adb_hbm.at[sidx.at[u, j, 1]],
                                  drows.at[b], sg.at[b]).wait()

        def issue_scatter(u, j, b):
            pltpu.async_copy(orow.at[b], accum.at[sidx.at[u, j, 2]],
                             so.at[b], add=True)

        def wait_scatter(u, j, b):
            pltpu.make_async_copy(orow.at[b], accum.at[sidx.at[u, j, 2]],
                                  so.at[b]).wait()

        def compute(b):
            @plsc.parallel_loop(0, K, 1, unroll=4)
            def edge_body(k):
                va = arows[b, k, pl.ds(0, 16)]
                vd = drows[b, k, pl.ds(0, 16)]
                u = _lrelu(va + vd)
                bsh = _vgather(vd, bidx)
                pv = jnp.exp(u - bsh)
                w0 = hrows[b, k, pl.ds(0, 16)] * _vgather(
                    pv, jnp.zeros((16,), dtype=jnp.int32))
                front = jnp.where(m8, pv, _vgather(w0, lanes & 7))
                orow[b, k, pl.ds(0, 16)] = front
                orow[b, k, pl.ds(8, 16)] = w0
                for j in range(1, nvec):
                    hj = hrows[b, k, pl.ds(16 * j, 16)]
                    pj = _vgather(pv, jnp.full((16,), (16 * j) // ch,
                                               dtype=jnp.int32))
                    orow[b, k, pl.ds(8 + 16 * j, 16)] = hj * pj

        pltpu.sync_copy(eidx_hbm.at[bbase], sidx.at[0])
        issue_gather(0, 0, 0)
        issue_idx(1, 1)

        def window_body(g, carry):
            for pos in range(8):
                ci = g * 8 + pos
                b = pos % 2
                u_cur = (pos // 4) % 2
                u_nxt = ((pos + 1) // 4) % 2
                u_pv2 = ((pos - 2) // 4) % 2

                @pl.when(ci + 1 < CH)
                def _():
                    if pos % 4 == 3:
                        wait_idx((ci + 1) // 4, u_nxt)
                    issue_gather(u_nxt, (pos + 1) % 4, 1 - b)

                wait_gather(u_cur, pos % 4, b)

                @pl.when(ci >= 2)
                def _():
                    wait_scatter(u_pv2, (pos - 2) % 4, b)

                if pos % 4 == 1:
                    blk_t = (ci + 3) // 4

                    @pl.when((ci >= 5) & (blk_t < chb))
                    def _():
                        issue_idx(blk_t, 1 if pos == 1 else 0)

                compute(b)
                issue_scatter(u_cur, pos % 4, b)
            return carry

        lax.fori_loop(0, CH // 8, window_body, 0, unroll=False)
        _ub = ((CH - 2) // 4) % 2
        wait_scatter(_ub, 2, 0)
        wait_scatter(_ub, 3, 1)
        plsc.subcore_barrier()
        pltpu.sync_copy(accum.at[pl.ds(r0, _RPT), pl.ds(8, hw)],
                        ow_hbm.at[c, pl.ds(r0, _RPT)])
        pltpu.sync_copy(accum.at[pl.ds(r0, _RPT), pl.ds(0, 8)],
                        op_hbm.at[c, pl.ds(r0, _RPT)])

    return sc_edge


_sc_edge1 = _make_sc_edge(_D, _H1, _C1, _K1, _CH1)
_sc_edge2 = _make_sc_edge(_C2, 1, _C2, _K2, _CH2)


def _block_diag(a):
    heads, ch = a.shape
    eye = jnp.eye(heads, dtype=jnp.float32)
    return (a[:, :, None] * eye[:, None, :]).reshape(heads * ch, heads)


def _build_eidx(e0, e1, K, EP):
    loops = jnp.arange(_N, dtype=jnp.int32)
    npad = EP - _EED
    ar = jnp.arange(npad, dtype=jnp.int32)
    pad_g = ar % _N
    pad_s = _N + (ar % 16)
    gsrc = jnp.concatenate([e0, loops, pad_g])
    gdst = jnp.concatenate([e1, loops, pad_g])
    sdst = jnp.concatenate([e1, loops, pad_s])
    ncht = EP // K
    st = jnp.stack([gsrc.reshape(ncht, K), gdst.reshape(ncht, K),
                    sdst.reshape(ncht, K)], axis=1)
    return st.reshape(ncht // 4, 4, 3, K)


def kernel(x, edge_index, W1, a_src1, a_dst1, b1, W2, a_src2, a_dst2, b2):
    eidx1 = _build_eidx(edge_index[0], edge_index[1], _K1, _EP1)
    eidx2 = _build_eidx(edge_index[0], edge_index[1], _K2, _EP2)

    asm1 = _block_diag(a_src1)
    adm1 = _block_diag(a_dst1)
    h1, asp1, adb1 = pl.pallas_call(
        _tc_pre1_body,
        out_shape=[
            jax.ShapeDtypeStruct((_N, _D), jnp.float32),
            jax.ShapeDtypeStruct((_N, _TROW), jnp.float32),
            jax.ShapeDtypeStruct((_N, _TROW), jnp.float32),
        ],
    )(x, W1, asm1, adm1)

    accw1, accp1 = _sc_edge1(h1, asp1, adb1, eidx1)

    rep = jnp.repeat(jnp.eye(_H1, dtype=jnp.float32), _C1, axis=1)
    h2, asp2, adb2 = pl.pallas_call(
        _tc_mid_body,
        out_shape=[
            jax.ShapeDtypeStruct((_N, _C2), jnp.float32),
            jax.ShapeDtypeStruct((_N, _TROW), jnp.float32),
            jax.ShapeDtypeStruct((_N, _TROW), jnp.float32),
        ],
    )(accw1, accp1, b1.reshape(1, _D), rep, W2, a_src2.reshape(_C2, 1),
      a_dst2.reshape(_C2, 1))

    accw2, accp2 = _sc_edge2(h2, asp2, adb2, eidx2)

    out = pl.pallas_call(
        _tc_post2_body,
        out_shape=jax.ShapeDtypeStruct((_N, _C2), jnp.float32),
    )(accw2, accp2, b2.reshape(1, _C2))
    return out
